# baseline (speedup 1.0000x reference)
import functools

import jax
import jax.numpy as jnp
from jax import lax
from jax.experimental import pallas as pl
from jax.experimental.pallas import tpu as pltpu
from jax.experimental.pallas import tpu_sc as plsc

_NC = 2
_NS = 16
_NW = _NC * _NS
_L = 16


def _pool_sc(x2d, idx5f, w_b, *, n_chunk, unroll):
    BT, C = x2d.shape
    BK = w_b.shape[0] // (5 * _L)
    rows_per_w = BK // _NW
    chunks = rows_per_w // n_chunk
    n5 = 5 * n_chunk
    assert rows_per_w % n_chunk == 0 and n5 % 8 == 0

    mesh = plsc.VectorSubcoreMesh(
        core_axis_name="c", subcore_axis_name="s", num_cores=_NC, num_subcores=_NS
    )

    vm = lambda shape, dt: pltpu.VMEM(shape, dt)

    @functools.partial(
        pl.kernel,
        out_type=jax.ShapeDtypeStruct((BK, C), jnp.float32),
        mesh=mesh,
        scratch_types=[
            vm((5 * rows_per_w,), jnp.int32),
            vm((rows_per_w * 5 * _L,), jnp.float32),
            [vm((n5, C), jnp.float32) for _ in range(2)],
            [vm((n_chunk, C), jnp.float32) for _ in range(2)],
            [pltpu.SemaphoreType.DMA for _ in range(2)],
            [pltpu.SemaphoreType.DMA for _ in range(2)],
        ],
    )
    def pool_kernel(x_hbm, idx_hbm, w_hbm, out_hbm, idx_v, w_v, buf_v, out_v, gsem, osem):
        wid = lax.axis_index("s") * _NC + lax.axis_index("c")
        base_w = wid * rows_per_w
        pltpu.sync_copy(idx_hbm.at[pl.ds(base_w * 5, rows_per_w * 5)], idx_v)
        pltpu.sync_copy(w_hbm.at[pl.ds(base_w * 5 * _L, rows_per_w * 5 * _L)], w_v)

        def gather(cc, sl):
            return pltpu.async_copy(
                x_hbm.at[idx_v.at[pl.ds(cc * n5, n5)]], buf_v[sl], gsem[sl]
            )

        gcp = [gather(0, 0), None]
        ocp = [None, None]
        for cc in range(chunks):
            cur = cc & 1
            nxt = cur ^ 1
            if cc + 1 < chunks:
                gcp[nxt] = gather(cc + 1, nxt)
            gcp[cur].wait()
            if ocp[cur] is not None:
                ocp[cur].wait()
            buf = buf_v[cur]
            out = out_v[cur]
            for q in range(n_chunk):
                w0 = w_v[pl.ds((cc * n_chunk + q) * 5 * _L + 0 * _L, _L)]
                w1 = w_v[pl.ds((cc * n_chunk + q) * 5 * _L + 1 * _L, _L)]
                w2 = w_v[pl.ds((cc * n_chunk + q) * 5 * _L + 2 * _L, _L)]
                w3 = w_v[pl.ds((cc * n_chunk + q) * 5 * _L + 3 * _L, _L)]
                w4 = w_v[pl.ds((cc * n_chunk + q) * 5 * _L + 4 * _L, _L)]

                @plsc.parallel_loop(0, C // _L, step=1, unroll=unroll)
                def c_body(c):
                    s = pl.ds(c * _L, _L)
                    r = w0 * buf[5 * q + 0, s]
                    r += w1 * buf[5 * q + 1, s]
                    r += w2 * buf[5 * q + 2, s]
                    r += w3 * buf[5 * q + 3, s]
                    r += w4 * buf[5 * q + 4, s]
                    out[q, s] = r

            base = base_w + cc * n_chunk
            ocp[cur] = pltpu.async_copy(out, out_hbm.at[pl.ds(base, n_chunk)], osem[cur])
        for sl in range(2):
            if ocp[sl] is not None:
                ocp[sl].wait()

    return pool_kernel(x2d, idx5f, w_b)


def _mlp_body(a_ref, w1_ref, b1_ref, w2_ref, b2_ref, o_ref):
    j = pl.program_id(1)
    h = jnp.dot(a_ref[...], w1_ref[...], preferred_element_type=jnp.float32)
    h = jnp.maximum(h + b1_ref[...], 0.0)
    p = jnp.dot(h, w2_ref[...], preferred_element_type=jnp.float32)

    @pl.when(j == 0)
    def _():
        o_ref[...] = p + b2_ref[...]

    @pl.when(j > 0)
    def _():
        o_ref[...] += p


def _mlp_tc(pooled, W1, b1, W2, b2, *, bm, bh):
    M, C = pooled.shape
    H = W1.shape[1]
    grid = (M // bm, H // bh)
    return pl.pallas_call(
        _mlp_body,
        grid=grid,
        in_specs=[
            pl.BlockSpec((bm, C), lambda i, j: (i, 0)),
            pl.BlockSpec((C, bh), lambda i, j: (0, j)),
            pl.BlockSpec((bh,), lambda i, j: (j,)),
            pl.BlockSpec((bh, 1), lambda i, j: (j, 0)),
            pl.BlockSpec((1,), lambda i, j: (0,)),
        ],
        out_specs=pl.BlockSpec((bm, 1), lambda i, j: (i, 0)),
        out_shape=jax.ShapeDtypeStruct((M, 1), jnp.float32),
        compiler_params=pltpu.CompilerParams(
            dimension_semantics=("parallel", "arbitrary"),
        ),
    )(pooled, W1, b1, W2, b2)


def kernel(x, coarse_ids, W1, b1, W2, b2):
    B, T, C = x.shape
    K = coarse_ids.shape[1]
    BK = B * K

    ids = coarse_ids.reshape(BK).astype(jnp.int32)
    boff = (jnp.arange(BK, dtype=jnp.int32) // K) * T
    offs = jnp.arange(-2, 3, dtype=jnp.int32)
    pos = ids[:, None] + offs[None, :]
    valid = (pos >= 0) & (pos < T)
    posc = jnp.clip(pos, 0, T - 1)
    idx5f = (boff[:, None] + posc).reshape(BK * 5)
    count = valid.sum(axis=1).astype(jnp.float32)
    w = valid.astype(jnp.float32) / count[:, None]
    w_b = jnp.broadcast_to(w[:, :, None], (BK, 5, _L)).reshape(BK * 5 * _L)

    pooled = _pool_sc(x.reshape(B * T, C), idx5f, w_b, n_chunk=8, unroll=8)
    out = _mlp_tc(pooled, W1, b1, W2, b2, bm=BK, bh=512)
    return out.reshape(B, K)


# speedup vs baseline: 1.0326x; 1.0326x over previous
import functools

import jax
import jax.numpy as jnp
from jax import lax
from jax.experimental import pallas as pl
from jax.experimental.pallas import tpu as pltpu
from jax.experimental.pallas import tpu_sc as plsc

_NC = 2
_NS = 16
_NW = _NC * _NS
_L = 16


def _pool_sc(x2d, idx5f, w_b, *, n_chunk, unroll):
    BT, C = x2d.shape
    BK = w_b.shape[0] // (5 * _L)
    rows_per_w = BK // _NW
    chunks = rows_per_w // n_chunk
    n5 = 5 * n_chunk
    assert rows_per_w % n_chunk == 0 and n5 % 8 == 0

    mesh = plsc.VectorSubcoreMesh(
        core_axis_name="c", subcore_axis_name="s", num_cores=_NC, num_subcores=_NS
    )

    vm = lambda shape, dt: pltpu.VMEM(shape, dt)

    @functools.partial(
        pl.kernel,
        out_type=jax.ShapeDtypeStruct((BK, C), jnp.float32),
        mesh=mesh,
        scratch_types=[
            vm((5 * rows_per_w,), jnp.int32),
            vm((rows_per_w * 5 * _L,), jnp.float32),
            [vm((n5, C), jnp.float32) for _ in range(2)],
            [vm((n_chunk, C), jnp.float32) for _ in range(2)],
            [pltpu.SemaphoreType.DMA for _ in range(2)],
            [pltpu.SemaphoreType.DMA for _ in range(2)],
            [pltpu.SemaphoreType.DMA for _ in range(2)],
        ],
    )
    def pool_kernel(x_hbm, idx_hbm, w_hbm, out_hbm, idx_v, w_v, buf_v, out_v, gsem, gsem2, osem):
        wid = lax.axis_index("s") * _NC + lax.axis_index("c")
        base_w = wid * rows_per_w
        pltpu.sync_copy(idx_hbm.at[pl.ds(base_w * 5, rows_per_w * 5)], idx_v)
        pltpu.sync_copy(w_hbm.at[pl.ds(base_w * 5 * _L, rows_per_w * 5 * _L)], w_v)

        def gather(cc, sl):
            a = pltpu.async_copy(
                x_hbm.at[idx_v.at[pl.ds(cc * n5, 24)]],
                buf_v[sl].at[pl.ds(0, 24)], gsem[sl],
            )
            b = pltpu.async_copy(
                x_hbm.at[idx_v.at[pl.ds(cc * n5 + 24, n5 - 24)]],
                buf_v[sl].at[pl.ds(24, n5 - 24)], gsem2[sl],
            )
            return (a, b)

        gcp = [gather(0, 0), None]
        ocp = [None, None]
        for cc in range(chunks):
            cur = cc & 1
            nxt = cur ^ 1
            if cc + 1 < chunks:
                gcp[nxt] = gather(cc + 1, nxt)
            gcp[cur][0].wait()
            gcp[cur][1].wait()
            if ocp[cur] is not None:
                ocp[cur].wait()
            buf = buf_v[cur]
            out = out_v[cur]
            for q in range(n_chunk):
                w0 = w_v[pl.ds((cc * n_chunk + q) * 5 * _L + 0 * _L, _L)]
                w1 = w_v[pl.ds((cc * n_chunk + q) * 5 * _L + 1 * _L, _L)]
                w2 = w_v[pl.ds((cc * n_chunk + q) * 5 * _L + 2 * _L, _L)]
                w3 = w_v[pl.ds((cc * n_chunk + q) * 5 * _L + 3 * _L, _L)]
                w4 = w_v[pl.ds((cc * n_chunk + q) * 5 * _L + 4 * _L, _L)]

                @plsc.parallel_loop(0, C // _L, step=1, unroll=unroll)
                def c_body(c):
                    s = pl.ds(c * _L, _L)
                    r = w0 * buf[5 * q + 0, s]
                    r += w1 * buf[5 * q + 1, s]
                    r += w2 * buf[5 * q + 2, s]
                    r += w3 * buf[5 * q + 3, s]
                    r += w4 * buf[5 * q + 4, s]
                    out[q, s] = r

            base = base_w + cc * n_chunk
            ocp[cur] = pltpu.async_copy(out, out_hbm.at[pl.ds(base, n_chunk)], osem[cur])
        for sl in range(2):
            if ocp[sl] is not None:
                ocp[sl].wait()

    return pool_kernel(x2d, idx5f, w_b)


def _mlp_body(a_ref, w1_ref, b1_ref, w2_ref, b2_ref, o_ref):
    j = pl.program_id(1)
    h = jnp.dot(a_ref[...], w1_ref[...], preferred_element_type=jnp.float32)
    h = jnp.maximum(h + b1_ref[...], 0.0)
    p = jnp.dot(h, w2_ref[...], preferred_element_type=jnp.float32)

    @pl.when(j == 0)
    def _():
        o_ref[...] = p + b2_ref[...]

    @pl.when(j > 0)
    def _():
        o_ref[...] += p


def _mlp_tc(pooled, W1, b1, W2, b2, *, bm, bh):
    M, C = pooled.shape
    H = W1.shape[1]
    grid = (M // bm, H // bh)
    return pl.pallas_call(
        _mlp_body,
        grid=grid,
        in_specs=[
            pl.BlockSpec((bm, C), lambda i, j: (i, 0)),
            pl.BlockSpec((C, bh), lambda i, j: (0, j)),
            pl.BlockSpec((bh,), lambda i, j: (j,)),
            pl.BlockSpec((bh, 1), lambda i, j: (j, 0)),
            pl.BlockSpec((1,), lambda i, j: (0,)),
        ],
        out_specs=pl.BlockSpec((bm, 1), lambda i, j: (i, 0)),
        out_shape=jax.ShapeDtypeStruct((M, 1), jnp.float32),
        compiler_params=pltpu.CompilerParams(
            dimension_semantics=("parallel", "arbitrary"),
        ),
    )(pooled, W1, b1, W2, b2)


def kernel(x, coarse_ids, W1, b1, W2, b2):
    B, T, C = x.shape
    K = coarse_ids.shape[1]
    BK = B * K

    ids = coarse_ids.reshape(BK).astype(jnp.int32)
    boff = (jnp.arange(BK, dtype=jnp.int32) // K) * T
    offs = jnp.arange(-2, 3, dtype=jnp.int32)
    pos = ids[:, None] + offs[None, :]
    valid = (pos >= 0) & (pos < T)
    posc = jnp.clip(pos, 0, T - 1)
    idx5f = (boff[:, None] + posc).reshape(BK * 5)
    count = valid.sum(axis=1).astype(jnp.float32)
    w = valid.astype(jnp.float32) / count[:, None]
    w_b = jnp.broadcast_to(w[:, :, None], (BK, 5, _L)).reshape(BK * 5 * _L)

    pooled = _pool_sc(x.reshape(B * T, C), idx5f, w_b, n_chunk=8, unroll=4)
    out = _mlp_tc(pooled, W1, b1, W2, b2, bm=BK, bh=512)
    return out.reshape(B, K)


# speedup vs baseline: 1.1370x; 1.1010x over previous
import functools

import jax
import jax.numpy as jnp
from jax import lax
from jax.experimental import pallas as pl
from jax.experimental.pallas import tpu as pltpu
from jax.experimental.pallas import tpu_sc as plsc

_NC = 2
_NS = 16
_NW = _NC * _NS
_L = 16


def _pool_sc(x2d, ids, *, n_chunk, unroll, K, T):
    BT, C = x2d.shape
    BK = ids.shape[0]
    assert K & (K - 1) == 0 and T & (T - 1) == 0
    _kb = K.bit_length() - 1
    _tb = T.bit_length() - 1
    rows_per_w = BK // _NW
    chunks = rows_per_w // n_chunk
    n5 = 5 * n_chunk
    assert rows_per_w % n_chunk == 0 and n5 % 8 == 0

    mesh = plsc.VectorSubcoreMesh(
        core_axis_name="c", subcore_axis_name="s", num_cores=_NC, num_subcores=_NS
    )

    vm = lambda shape, dt: pltpu.VMEM(shape, dt)

    @functools.partial(
        pl.kernel,
        out_type=jax.ShapeDtypeStruct((BK, C), jnp.float32),
        mesh=mesh,
        scratch_types=[
            vm((rows_per_w,), jnp.int32),
            vm((5 * rows_per_w,), jnp.int32),
            vm((5 * rows_per_w,), jnp.float32),
            [vm((n5, C), jnp.float32) for _ in range(2)],
            [vm((n_chunk, C), jnp.float32) for _ in range(2)],
            [pltpu.SemaphoreType.DMA for _ in range(2)],
            [pltpu.SemaphoreType.DMA for _ in range(2)],
            [pltpu.SemaphoreType.DMA for _ in range(2)],
        ],
    )
    def pool_kernel(x_hbm, ids_hbm, out_hbm, ids_v, idx_v, w_v, buf_v, out_v, gsem, gsem2, osem):
        wid = lax.axis_index("s") * _NC + lax.axis_index("c")
        base_w = wid * rows_per_w
        pltpu.sync_copy(ids_hbm.at[pl.ds(base_w, rows_per_w)], ids_v)
        lanes = lax.iota(jnp.int32, _L)
        for g in range(rows_per_w // _L):
            idsv = ids_v[pl.ds(g * _L, _L)]
            rowv = base_w + g * _L + lanes
            boffv = lax.shift_left(lax.shift_right_logical(rowv, _kb), _tb)
            valids = []
            cnt = None
            for o in range(5):
                posv = idsv + (o - 2)
                vmask = (posv >= 0) & (posv < T)
                valids.append(vmask)
                pc = jnp.clip(posv, 0, T - 1)
                idx_v[pl.ds(o * rows_per_w + g * _L, _L)] = boffv + pc
                iv = jnp.where(vmask, 1, 0)
                cnt = iv if cnt is None else cnt + iv
            invc = 1.0 / cnt.astype(jnp.float32)
            zero = jnp.zeros((_L,), jnp.float32)
            for o in range(5):
                w_v[pl.ds(o * rows_per_w + g * _L, _L)] = jnp.where(
                    valids[o], invc, zero
                )

        def gather(cc, sl):
            cps = []
            for o in range(5):
                cps.append(pltpu.async_copy(
                    x_hbm.at[idx_v.at[pl.ds(o * rows_per_w + cc * n_chunk, n_chunk)]],
                    buf_v[sl].at[pl.ds(o * n_chunk, n_chunk)], gsem[sl],
                ))
            return cps

        gcp = [gather(0, 0), None]
        ocp = [None, None]
        for cc in range(chunks):
            cur = cc & 1
            nxt = cur ^ 1
            if cc + 1 < chunks:
                gcp[nxt] = gather(cc + 1, nxt)
            for cp in gcp[cur]:
                cp.wait()
            if ocp[cur] is not None:
                ocp[cur].wait()
            buf = buf_v[cur]
            out = out_v[cur]
            for q in range(n_chunk):
                qg = cc * n_chunk + q
                g, lane = divmod(qg, _L)
                bidx = jnp.full((_L,), lane, jnp.int32)
                wv = []
                for o in range(5):
                    w16 = w_v[pl.ds(o * rows_per_w + g * _L, _L)]
                    wv.append(
                        lax.gather(
                            w16,
                            bidx[:, None],
                            lax.GatherDimensionNumbers(
                                offset_dims=(),
                                collapsed_slice_dims=(0,),
                                start_index_map=(0,),
                            ),
                            (1,),
                            mode=lax.GatherScatterMode.PROMISE_IN_BOUNDS,
                        )
                    )
                w0, w1, w2, w3, w4 = wv

                @plsc.parallel_loop(0, C // _L, step=1, unroll=unroll)
                def c_body(c):
                    s = pl.ds(c * _L, _L)
                    r = w0 * buf[q, s]
                    r += w1 * buf[n_chunk + q, s]
                    r += w2 * buf[2 * n_chunk + q, s]
                    r += w3 * buf[3 * n_chunk + q, s]
                    r += w4 * buf[4 * n_chunk + q, s]
                    out[q, s] = r

            base = base_w + cc * n_chunk
            ocp[cur] = pltpu.async_copy(out, out_hbm.at[pl.ds(base, n_chunk)], osem[cur])
        for sl in range(2):
            if ocp[sl] is not None:
                ocp[sl].wait()

    return pool_kernel(x2d, ids)


def _mlp_body(a_ref, w1_ref, b1_ref, w2_ref, b2_ref, o_ref):
    j = pl.program_id(1)
    h = jnp.dot(a_ref[...], w1_ref[...], preferred_element_type=jnp.float32)
    h = jnp.maximum(h + b1_ref[...], 0.0)
    p = jnp.dot(h, w2_ref[...], preferred_element_type=jnp.float32)

    @pl.when(j == 0)
    def _():
        o_ref[...] = p + b2_ref[...]

    @pl.when(j > 0)
    def _():
        o_ref[...] += p


def _mlp_tc(pooled, W1, b1, W2, b2, *, bm, bh):
    M, C = pooled.shape
    H = W1.shape[1]
    grid = (M // bm, H // bh)
    return pl.pallas_call(
        _mlp_body,
        grid=grid,
        in_specs=[
            pl.BlockSpec((bm, C), lambda i, j: (i, 0)),
            pl.BlockSpec((C, bh), lambda i, j: (0, j)),
            pl.BlockSpec((bh,), lambda i, j: (j,)),
            pl.BlockSpec((bh, 1), lambda i, j: (j, 0)),
            pl.BlockSpec((1,), lambda i, j: (0,)),
        ],
        out_specs=pl.BlockSpec((bm, 1), lambda i, j: (i, 0)),
        out_shape=jax.ShapeDtypeStruct((M, 1), jnp.float32),
        compiler_params=pltpu.CompilerParams(
            dimension_semantics=("parallel", "arbitrary"),
        ),
    )(pooled, W1, b1, W2, b2)


def kernel(x, coarse_ids, W1, b1, W2, b2):
    B, T, C = x.shape
    K = coarse_ids.shape[1]
    BK = B * K

    ids = coarse_ids.reshape(BK)
    pooled = _pool_sc(x.reshape(B * T, C), ids, n_chunk=8, unroll=4, K=K, T=T)
    out = _mlp_tc(pooled, W1, b1, W2, b2, bm=BK, bh=512)
    return out.reshape(B, K)
